# baseline (speedup 1.0000x reference)
import functools

import jax
import jax.numpy as jnp
from jax import lax
from jax.experimental import pallas as pl
from jax.experimental.pallas import tpu as pltpu
from jax.experimental.pallas import tpu_sc as plsc


def _project_body(table_ref, w_ref, b_ref, out_ref):
    out_ref[...] = (
        lax.dot_general(
            table_ref[...],
            w_ref[...],
            (((1,), (1,)), ((), ())),
            preferred_element_type=jnp.float32,
        )
        + b_ref[...]
    )


def _project(table_padded, W, b2d):
    vp, d = table_padded.shape
    return pl.pallas_call(
        _project_body,
        out_shape=jax.ShapeDtypeStruct((vp, d), jnp.float32),
    )(table_padded, W, b2d)


def _make_gather(v, d, batch):
    info = plsc.get_sparse_core_info()
    nc, ns = info.num_cores, info.num_subcores
    nw = nc * ns
    assert batch % nw == 0
    b_per_w = batch // nw

    mesh = plsc.VectorSubcoreMesh(core_axis_name="c", subcore_axis_name="s")

    assert (v * d) % ns == 0
    slice_w = (v * d) // ns

    @functools.partial(
        pl.kernel,
        mesh=mesh,
        out_type=jax.ShapeDtypeStruct((batch * d,), jnp.float32),
        scratch_types=[
            pltpu.VMEM((b_per_w,), jnp.int32),
            pltpu.VMEM((slice_w,), jnp.float32),
            pltpu.VMEM_SHARED((v * d,), jnp.float32),
            pltpu.SemaphoreType.DMA,
        ],
    )
    def gather_kernel(proj_hbm, idx_hbm, out_hbm, idx_s, stage_v, tab_v, wsem):
        sid = lax.axis_index("s")
        wid = sid * nc + lax.axis_index("c")
        base = wid * b_per_w

        pltpu.sync_copy(proj_hbm.at[pl.ds(sid * slice_w, slice_w)], stage_v)
        pltpu.sync_copy(stage_v, tab_v.at[pl.ds(sid * slice_w, slice_w)])
        pltpu.sync_copy(idx_hbm.at[pl.ds(base, b_per_w)], idx_s)
        plsc.subcore_barrier()

        n_groups = b_per_w // 16

        def body(jo, carry):
            ids16 = idx_s[pl.ds(jo * 16, 16)]
            for k in range(16):
                row = ids16[k]
                pltpu.async_copy(
                    tab_v.at[pl.ds(row * d, d)],
                    out_hbm.at[pl.ds((base + jo * 16 + k) * d, d)],
                    wsem,
                )
            return carry

        lax.fori_loop(0, n_groups, body, 0)
        for _ in range(n_groups):
            pltpu.make_async_copy(
                tab_v.at[pl.ds(0, 16 * d)],
                out_hbm.at[pl.ds(base * d, 16 * d)],
                wsem,
            ).wait()

    return gather_kernel


def kernel(expert_id, table, W, b):
    v, d = table.shape
    (batch,) = expert_id.shape
    vp = (v + 7) // 8 * 8
    table_padded = jnp.pad(table, ((0, vp - v), (0, 0)))
    projected = _project(table_padded, W, b.reshape(1, d))[:v]
    ids = expert_id.astype(jnp.int32)
    out = _make_gather(v, d, batch)(projected.reshape(-1), ids)
    return out.reshape(batch, d)


# speedup vs baseline: 1.9890x; 1.9890x over previous
import functools

import jax
import jax.numpy as jnp
from jax import lax
from jax.experimental import pallas as pl
from jax.experimental.pallas import tpu as pltpu
from jax.experimental.pallas import tpu_sc as plsc


def _project_body(table_ref, w_ref, b_ref, out_ref):
    out_ref[...] = (
        lax.dot_general(
            table_ref[...],
            w_ref[...],
            (((1,), (1,)), ((), ())),
            preferred_element_type=jnp.float32,
        )
        + b_ref[...]
    )


def _project(table_padded, W, b2d):
    vp, d = table_padded.shape
    return pl.pallas_call(
        _project_body,
        out_shape=jax.ShapeDtypeStruct((vp, d), jnp.float32),
    )(table_padded, W, b2d)


def _make_gather(v, d, batch):
    info = plsc.get_sparse_core_info()
    nc, ns = info.num_cores, info.num_subcores
    nw = nc * ns
    assert batch % nw == 0
    b_per_w = batch // nw

    mesh = plsc.VectorSubcoreMesh(core_axis_name="c", subcore_axis_name="s")

    @functools.partial(
        pl.kernel,
        mesh=mesh,
        out_type=jax.ShapeDtypeStruct((batch, d), jnp.float32),
        scratch_types=[
            pltpu.VMEM((b_per_w,), jnp.int32),
            pltpu.VMEM((v, d), jnp.float32),
            pltpu.SemaphoreType.DMA,
            pltpu.SemaphoreType.DMA,
        ],
    )
    def gather_kernel(proj_hbm, idx_hbm, out_hbm, idx_s, tab_v, tsem, wsem):
        wid = lax.axis_index("s") * nc + lax.axis_index("c")
        base = wid * b_per_w
        tstage = pltpu.async_copy(proj_hbm, tab_v, tsem)
        pltpu.sync_copy(idx_hbm.at[pl.ds(base, b_per_w)], idx_s)
        tstage.wait()

        n_groups = b_per_w // 16

        def body(jo, carry):
            ids16 = idx_s[pl.ds(jo * 16, 16)]
            for k in range(16):
                row = ids16[k]
                pltpu.async_copy(
                    tab_v.at[row],
                    out_hbm.at[base + jo * 16 + k],
                    wsem,
                )
            return carry

        lax.fori_loop(0, n_groups, body, 0)
        for _ in range(n_groups):
            pltpu.make_async_copy(
                tab_v.at[pl.ds(0, 16)],
                out_hbm.at[pl.ds(base, 16)],
                wsem,
            ).wait()

    return gather_kernel


def kernel(expert_id, table, W, b):
    v, d = table.shape
    (batch,) = expert_id.shape
    vp = (v + 7) // 8 * 8
    table_padded = jnp.pad(table, ((0, vp - v), (0, 0)))
    projected = _project(table_padded, W, b.reshape(1, d))[:v]
    ids = expert_id.astype(jnp.int32)
    out = _make_gather(v, d, batch)(projected, ids)
    return out


# speedup vs baseline: 2.0023x; 1.0067x over previous
import functools

import jax
import jax.numpy as jnp
from jax import lax
from jax.experimental import pallas as pl
from jax.experimental.pallas import tpu as pltpu
from jax.experimental.pallas import tpu_sc as plsc


def _project_body(table_ref, w_ref, b_ref, out_ref):
    out_ref[...] = (
        lax.dot_general(
            table_ref[...],
            w_ref[...],
            (((1,), (1,)), ((), ())),
            preferred_element_type=jnp.float32,
        )
        + b_ref[...]
    )


def _project(table_padded, W, b2d):
    vp, d = table_padded.shape
    return pl.pallas_call(
        _project_body,
        out_shape=jax.ShapeDtypeStruct((vp, d), jnp.float32),
    )(table_padded, W, b2d)


def _make_gather(v, d, batch):
    info = plsc.get_sparse_core_info()
    nc, ns = info.num_cores, info.num_subcores
    nw = nc * ns
    assert batch % nw == 0
    b_per_w = batch // nw

    mesh = plsc.VectorSubcoreMesh(core_axis_name="c", subcore_axis_name="s")

    @functools.partial(
        pl.kernel,
        mesh=mesh,
        out_type=jax.ShapeDtypeStruct((batch, d), jnp.float32),
        scratch_types=[
            pltpu.VMEM((b_per_w,), jnp.int32),
            pltpu.VMEM((v, d), jnp.float32),
            pltpu.SemaphoreType.DMA,
            pltpu.SemaphoreType.DMA,
        ],
    )
    def gather_kernel(proj_hbm, idx_hbm, out_hbm, idx_s, tab_v, tsem, wsem):
        wid = lax.axis_index("s") * nc + lax.axis_index("c")
        base = wid * b_per_w
        tstage = pltpu.async_copy(proj_hbm, tab_v, tsem)
        pltpu.sync_copy(idx_hbm.at[pl.ds(base, b_per_w)], idx_s)
        tstage.wait()

        n_groups = b_per_w // 16

        def body(jo, carry):
            ids16 = idx_s[pl.ds(jo * 16, 16)]
            for k in range(16):
                row = ids16[k]
                pltpu.async_copy(
                    tab_v.at[row],
                    out_hbm.at[base + jo * 16 + k],
                    wsem,
                )
            return carry

        lax.fori_loop(0, n_groups, body, 0)
        for _ in range(n_groups):
            pltpu.make_async_copy(
                tab_v.at[pl.ds(0, 16)],
                out_hbm.at[pl.ds(base, 16)],
                wsem,
            ).wait()

    return gather_kernel


def kernel(expert_id, table, W, b):
    v, d = table.shape
    (batch,) = expert_id.shape
    vp = (v + 7) // 8 * 8
    table_padded = jnp.pad(table, ((0, vp - v), (0, 0)))
    projected = _project(table_padded, W, b.reshape(1, d))
    ids = expert_id.astype(jnp.int32)
    out = _make_gather(vp, d, batch)(projected, ids)
    return out


# speedup vs baseline: 2.0494x; 1.0235x over previous
import functools

import jax
import jax.numpy as jnp
from jax import lax
from jax.experimental import pallas as pl
from jax.experimental.pallas import tpu as pltpu
from jax.experimental.pallas import tpu_sc as plsc


def _project_body(table_ref, w_ref, b_ref, out_ref):
    out_ref[...] = (
        lax.dot_general(
            table_ref[...],
            w_ref[...],
            (((1,), (1,)), ((), ())),
            preferred_element_type=jnp.float32,
        )
        + b_ref[...]
    )


def _project(table_padded, W, b2d):
    vp, d = table_padded.shape
    return pl.pallas_call(
        _project_body,
        out_shape=jax.ShapeDtypeStruct((vp, d), jnp.float32),
    )(table_padded, W, b2d)


def _make_gather(v, d, batch):
    info = plsc.get_sparse_core_info()
    nc, ns = info.num_cores, info.num_subcores
    nw = nc * ns
    assert batch % nw == 0
    b_per_w = batch // nw

    mesh = plsc.VectorSubcoreMesh(core_axis_name="c", subcore_axis_name="s")

    @functools.partial(
        pl.kernel,
        mesh=mesh,
        out_type=jax.ShapeDtypeStruct((batch, d), jnp.float32),
        scratch_types=[
            pltpu.VMEM((b_per_w,), jnp.int32),
            pltpu.VMEM((v, d), jnp.float32),
            pltpu.SemaphoreType.DMA,
            pltpu.SemaphoreType.DMA,
        ],
    )
    def gather_kernel(proj_hbm, idx_hbm, out_hbm, idx_s, tab_v, tsem, wsem):
        wid = lax.axis_index("s") * nc + lax.axis_index("c")
        base = wid * b_per_w
        tstage = pltpu.async_copy(proj_hbm, tab_v, tsem)
        pltpu.sync_copy(idx_hbm.at[pl.ds(base, b_per_w)], idx_s)
        tstage.wait()

        n_groups = b_per_w // 16

        def body(jo, carry):
            ids16 = idx_s[pl.ds(jo * 16, 16)]
            for k in range(16):
                row = ids16[k]
                pltpu.async_copy(
                    tab_v.at[row],
                    out_hbm.at[base + jo * 16 + k],
                    wsem,
                )
            return carry

        lax.fori_loop(0, n_groups, body, 0)
        for _ in range(n_groups):
            pltpu.make_async_copy(
                tab_v.at[pl.ds(0, 16)],
                out_hbm.at[pl.ds(base, 16)],
                wsem,
            ).wait()

    return gather_kernel


def kernel(expert_id, table, W, b):
    v, d = table.shape
    (batch,) = expert_id.shape
    projected = _project(table, W, b.reshape(1, d))
    ids = expert_id.astype(jnp.int32)
    out = _make_gather(v, d, batch)(projected, ids)
    return out
